# Initial kernel scaffold; baseline (speedup 1.0000x reference)
import jax
import jax.numpy as jnp
from jax.experimental import pallas as pl


def kernel(slots):
    raise NotImplementedError("write your pallas kernel here")


# speedup vs baseline: 2608.3730x; 2608.3730x over previous
import jax
import jax.numpy as jnp
from jax.experimental import pallas as pl

_THRESHOLD = 0.9


def _merge_body(slots_ref, merged_ref, mt_ref):
    n = slots_ref.shape[1]
    x = slots_ref[0]
    nrm = jnp.sqrt(jnp.sum(x * x, axis=1, keepdims=True))
    xn = x / jnp.maximum(nrm, 1e-12)
    sim = jax.lax.dot_general(
        xn, xn, (((1,), (1,)), ((), ())),
        preferred_element_type=jnp.float32,
        precision=jax.lax.Precision.HIGHEST,
    )
    row_i = jax.lax.broadcasted_iota(jnp.int32, (n, n), 0)
    col_i = jax.lax.broadcasted_iota(jnp.int32, (n, n), 1)
    sim = jnp.where(row_i == col_i, sim - 2.0, sim)
    flat_i = row_i * n + col_i

    col1 = jax.lax.broadcasted_iota(jnp.int32, (1, n), 1)
    rowN = jax.lax.broadcasted_iota(jnp.int32, (n, 1), 0)

    def cond(st):
        return st[3] != 0

    def body(st):
        alive_r, alive_c, mt, _ = st
        masked = jnp.where(alive_r * alive_c > 0.5, sim, -2.0)
        rowmax = jnp.max(masked, axis=1, keepdims=True)
        m = jnp.max(rowmax)
        go = m > _THRESHOLD
        cand = jnp.where(masked == m, flat_i, jnp.int32(0x7FFFFFFF))
        fi = jnp.min(cand)
        r = fi // n
        c = fi - r * n
        src = jnp.maximum(r, c)
        tgt = jnp.minimum(r, c)
        mt = jnp.where(go & (col1 == src), tgt, mt)
        alive_r = jnp.where(go & (rowN == src), 0.0, alive_r)
        alive_c = jnp.where(go & (col1 == src), 0.0, alive_c)
        return (alive_r, alive_c, mt, go.astype(jnp.int32))

    init = (
        jnp.ones((n, 1), dtype=jnp.float32),
        jnp.ones((1, n), dtype=jnp.float32),
        col1,
        jnp.int32(1),
    )
    _, _, mt, _ = jax.lax.while_loop(cond, body, init)

    m_t = (rowN == mt).astype(jnp.float32)
    merged = jax.lax.dot_general(
        m_t, x, (((1,), (0,)), ((), ())),
        preferred_element_type=jnp.float32,
        precision=jax.lax.Precision.HIGHEST,
    )
    counts = jnp.sum(m_t, axis=1, keepdims=True)
    merged_ref[0] = merged / jnp.maximum(counts, 1.0)
    mt_ref[0] = mt


def kernel(slots):
    b, n, d = slots.shape
    merged, mt3 = pl.pallas_call(
        _merge_body,
        grid=(b,),
        in_specs=[pl.BlockSpec((1, n, d), lambda i: (i, 0, 0))],
        out_specs=[
            pl.BlockSpec((1, n, d), lambda i: (i, 0, 0)),
            pl.BlockSpec((1, 1, n), lambda i: (i, 0, 0)),
        ],
        out_shape=[
            jax.ShapeDtypeStruct((b, n, d), jnp.float32),
            jax.ShapeDtypeStruct((b, 1, n), jnp.int32),
        ],
    )(slots)
    return merged, mt3.reshape(b, n)
